# baseline (speedup 1.0000x reference)
import functools

import jax
import jax.numpy as jnp
from jax import lax
from jax.experimental import pallas as pl
from jax.experimental.pallas import tpu as pltpu
from jax.experimental.pallas import tpu_sc as plsc

N = 10000
D = 128
K = 16
NPAD = 10240
RQ = 64

P_LVL = 4
NW = 32
NPW = NPAD // NW
BN = 8
NIT = NPW // BN


def _knn_body(featq_ref, featT_ref, wtT_ref, wpT_ref, bsum_ref,
              idx_ref, t0_ref, base_ref, dist_ref):
    q = featq_ref[...]
    kT = featT_ref[...]
    t0 = jnp.dot(q, wtT_ref[...], preferred_element_type=jnp.float32)
    ph = jnp.dot(q, wpT_ref[...], preferred_element_type=jnp.float32)
    t0_ref[...] = t0
    base_ref[...] = t0 + ph + bsum_ref[...]

    sqk = jnp.sum(kT * kT, axis=0, keepdims=True)
    colv = lax.broadcasted_iota(jnp.int32, (1, NPAD), 1)
    d = sqk - 2.0 * jnp.dot(q, kT, preferred_element_type=jnp.float32)
    dist_ref[...] = jnp.where(colv >= N, jnp.float32(1e30), d)

    NC = NPAD // 16
    INF = jnp.float32(jnp.inf)
    BIGI = jnp.int32(2 ** 30)

    def levels_after(vlo, clo):
        dd3 = dist_ref[...].reshape(RQ, 16, NC)
        s3 = lax.broadcasted_iota(jnp.int32, (RQ, 16, NC), 1)
        c3 = lax.broadcasted_iota(jnp.int32, (RQ, 16, NC), 2)
        col3 = s3 * NC + c3
        cms, ccs = [], []
        for _ in range(P_LVL):
            mask = (dd3 > vlo) | ((dd3 == vlo) & (col3 > clo))
            dm = jnp.where(mask, dd3, INF)
            cm = jnp.min(dm, axis=1)
            cc = jnp.min(jnp.where(dm == cm[:, None, :], col3, BIGI),
                         axis=1)
            cms.append(cm)
            ccs.append(cc)
            vlo, clo = cm[:, None, :], cc[:, None, :]
        return tuple(cms), tuple(ccs)

    def extract(t, carry):
        cms, ccs, acc = carry
        cm1 = cms[0]
        mt = jnp.min(cm1, axis=1, keepdims=True)
        it = jnp.min(jnp.where(cm1 == mt, ccs[0], BIGI),
                     axis=1, keepdims=True)
        acc = jnp.where(
            lax.broadcasted_iota(jnp.int32, (RQ, K), 1) == t, it, acc)
        sel = (cm1 == mt) & (ccs[0] == it)
        blind = jnp.any(sel & (cms[1] == INF))

        def promote(_):
            nvs = [jnp.where(sel, cms[k + 1], cms[k])
                   for k in range(P_LVL - 1)]
            ncs = [jnp.where(sel, ccs[k + 1], ccs[k])
                   for k in range(P_LVL - 1)]
            nvs.append(jnp.where(sel, INF, cms[-1]))
            ncs.append(jnp.where(sel, BIGI, ccs[-1]))
            return tuple(nvs), tuple(ncs)

        def refill(_):
            return levels_after(mt[:, :, None], it[:, :, None])

        cms, ccs = lax.cond(blind, refill, promote, 0)
        return cms, ccs, acc

    cms0, ccs0 = levels_after(jnp.float32(-jnp.inf), jnp.int32(-1))
    acc0 = jnp.zeros((RQ, K), jnp.int32)
    _, _, acc = lax.fori_loop(0, K, extract, (cms0, ccs0, acc0))
    idx_ref[...] = acc


def _knn_call(featp, featT, wtT, wpT, bsum):
    return pl.pallas_call(
        _knn_body,
        grid=(NPAD // RQ,),
        in_specs=[
            pl.BlockSpec((RQ, D), lambda i: (i, 0)),
            pl.BlockSpec((D, NPAD), lambda i: (0, 0)),
            pl.BlockSpec((D, D), lambda i: (0, 0)),
            pl.BlockSpec((D, D), lambda i: (0, 0)),
            pl.BlockSpec((1, D), lambda i: (0, 0)),
        ],
        out_specs=[
            pl.BlockSpec((RQ, K), lambda i: (i, 0)),
            pl.BlockSpec((RQ, D), lambda i: (i, 0)),
            pl.BlockSpec((RQ, D), lambda i: (i, 0)),
        ],
        out_shape=[
            jax.ShapeDtypeStruct((NPAD, K), jnp.int32),
            jax.ShapeDtypeStruct((NPAD, D), jnp.float32),
            jax.ShapeDtypeStruct((NPAD, D), jnp.float32),
        ],
        scratch_shapes=[pltpu.VMEM((RQ, NPAD), jnp.float32)],
    )(featp, featT, wtT, wpT, bsum)


def _sc_body(t0_hbm, base_hbm, idx_hbm, out_hbm, idxv, rows, basev, outv, sem):
    nc = plsc.get_sparse_core_info().num_cores
    wid = lax.axis_index("s") * nc + lax.axis_index("c")

    def body(g, carry):
        nb = wid * NPW + g * BN
        pltpu.sync_copy(idx_hbm.at[pl.ds(nb * K, BN * K)], idxv)
        pltpu.async_copy(t0_hbm.at[idxv], rows, sem).wait()
        pltpu.sync_copy(base_hbm.at[pl.ds(nb, BN)], basev)
        for b in range(BN):
            for c in range(D // 16):
                sl = pl.ds(c * 16, 16)
                acc = rows[b * K, sl]
                for j in range(1, K):
                    acc = jnp.minimum(acc, rows[b * K + j, sl])
                outv[b, sl] = basev[b, sl] - acc
        pltpu.sync_copy(outv, out_hbm.at[pl.ds(nb, BN)])
        return carry

    lax.fori_loop(0, NIT, body, 0)


@functools.cache
def _sc_gather_min():
    return pl.kernel(
        _sc_body,
        out_type=jax.ShapeDtypeStruct((NPAD, D), jnp.float32),
        mesh=plsc.VectorSubcoreMesh(core_axis_name="c", subcore_axis_name="s"),
        scratch_types=[
            pltpu.VMEM((BN * K,), jnp.int32),
            pltpu.VMEM((BN * K, D), jnp.float32),
            pltpu.VMEM((BN, D), jnp.float32),
            pltpu.VMEM((BN, D), jnp.float32),
            pltpu.SemaphoreType.DMA,
        ],
    )


@jax.jit
def kernel(feat, W_theta, b_theta, W_phi, b_phi):
    featp = jnp.pad(feat, ((0, NPAD - N), (0, 0)))
    featT = featp.T
    bsum = (b_theta + b_phi).reshape(1, D)
    idx, t0, base = _knn_call(featp, featT, W_theta.T, W_phi.T, bsum)
    out = _sc_gather_min()(t0, base, idx.reshape(-1))
    return out[:N]


# speedup vs baseline: 3.1899x; 3.1899x over previous
import functools

import jax
import jax.numpy as jnp
from jax import lax
from jax.experimental import pallas as pl
from jax.experimental.pallas import tpu as pltpu
from jax.experimental.pallas import tpu_sc as plsc

N = 10000
D = 128
K = 16
NPAD = 10240
RQ = 64

P_LVL = 4
NW = 32
NPW = NPAD // NW
BN = 8
NIT = NPW // BN


def _dist_block(featq_ref, featT_ref, dist_ref):
    q = featq_ref[...]
    kT = featT_ref[...]
    sqk = jnp.sum(kT * kT, axis=0, keepdims=True)
    colv = lax.broadcasted_iota(jnp.int32, (1, NPAD), 1)
    d = sqk - 2.0 * jnp.dot(q, kT, preferred_element_type=jnp.float32)
    d = jnp.where(colv >= N, jnp.float32(1e30), d)
    dist_ref[...] = d.reshape(RQ, 16, NPAD // 16)


def _knn_body(featq_ref, featT_ref, wtT_ref, wpT_ref, bsum_ref,
              idx_ref, t0_ref, base_ref, ovf_ref, dist_ref):
    q = featq_ref[...]
    t0 = jnp.dot(q, wtT_ref[...], preferred_element_type=jnp.float32)
    ph = jnp.dot(q, wpT_ref[...], preferred_element_type=jnp.float32)
    t0_ref[...] = t0
    base_ref[...] = t0 + ph + bsum_ref[...]
    _dist_block(featq_ref, featT_ref, dist_ref)

    NC = NPAD // 16
    INF = jnp.float32(jnp.inf)
    BIGI = jnp.int32(2 ** 30)
    s3 = lax.broadcasted_iota(jnp.int32, (RQ, 16, NC), 1)
    c3 = lax.broadcasted_iota(jnp.int32, (RQ, 16, NC), 2)
    col3 = s3 * NC + c3

    def levels_after(vlo, clo):
        dd3 = dist_ref[...]
        cms, ccs = [], []
        for _ in range(P_LVL):
            mask = (dd3 > vlo) | ((dd3 == vlo) & (col3 > clo))
            dm = jnp.where(mask, dd3, INF)
            cm = jnp.min(dm, axis=1)
            cc = jnp.min(jnp.where(dm == cm[:, None, :], col3, BIGI),
                         axis=1)
            cms.append(cm)
            ccs.append(cc)
            vlo, clo = cm[:, None, :], cc[:, None, :]
        return tuple(cms), tuple(ccs)

    cms, ccs = levels_after(jnp.float32(-jnp.inf), jnp.int32(-1))
    picks = []
    ovf = jnp.zeros((RQ, 1), jnp.bool_)
    for _ in range(K):
        cm1 = cms[0]
        mt = jnp.min(cm1, axis=1, keepdims=True)
        it = jnp.min(jnp.where(cm1 == mt, ccs[0], BIGI),
                     axis=1, keepdims=True)
        picks.append(it)
        sel = (cm1 == mt) & (ccs[0] == it)
        ovf = ovf | jnp.any(sel & (cms[1] == INF), axis=1, keepdims=True)
        nvs = [jnp.where(sel, cms[k + 1], cms[k]) for k in range(P_LVL - 1)]
        ncs = [jnp.where(sel, ccs[k + 1], ccs[k]) for k in range(P_LVL - 1)]
        nvs.append(jnp.where(sel, INF, cms[-1]))
        ncs.append(jnp.where(sel, BIGI, ccs[-1]))
        cms, ccs = tuple(nvs), tuple(ncs)
    idx_ref[...] = jnp.concatenate(picks, axis=1)
    ovf_ref[...] = jnp.broadcast_to(ovf, (RQ, K)).astype(jnp.int32)


def _slow_body(featq_ref, featT_ref, idx_ref, dist_ref):
    _dist_block(featq_ref, featT_ref, dist_ref)
    NC = NPAD // 16
    INF = jnp.float32(jnp.inf)
    BIGI = jnp.int32(2 ** 30)
    s3 = lax.broadcasted_iota(jnp.int32, (RQ, 16, NC), 1)
    c3 = lax.broadcasted_iota(jnp.int32, (RQ, 16, NC), 2)
    col3 = s3 * NC + c3
    m = jnp.full((RQ, 1, 1), -jnp.inf, jnp.float32)
    i = jnp.full((RQ, 1, 1), -1, jnp.int32)
    outs = []
    for _ in range(K):
        dd3 = dist_ref[...]
        ok = (dd3 > m) | ((dd3 == m) & (col3 > i))
        dm = jnp.where(ok, dd3, INF)
        mt = jnp.min(jnp.min(dm, axis=1), axis=1)[:, None, None]
        it = jnp.min(jnp.min(jnp.where((dd3 == mt) & ok, col3, BIGI),
                             axis=1), axis=1)[:, None, None]
        outs.append(it[:, :, 0])
        m, i = mt, it
    idx_ref[...] = jnp.concatenate(outs, axis=1)


def _knn_call(featp, featT, wtT, wpT, bsum):
    return pl.pallas_call(
        _knn_body,
        grid=(NPAD // RQ,),
        in_specs=[
            pl.BlockSpec((RQ, D), lambda i: (i, 0)),
            pl.BlockSpec((D, NPAD), lambda i: (0, 0)),
            pl.BlockSpec((D, D), lambda i: (0, 0)),
            pl.BlockSpec((D, D), lambda i: (0, 0)),
            pl.BlockSpec((1, D), lambda i: (0, 0)),
        ],
        out_specs=[
            pl.BlockSpec((RQ, K), lambda i: (i, 0)),
            pl.BlockSpec((RQ, D), lambda i: (i, 0)),
            pl.BlockSpec((RQ, D), lambda i: (i, 0)),
            pl.BlockSpec((RQ, K), lambda i: (i, 0)),
        ],
        out_shape=[
            jax.ShapeDtypeStruct((NPAD, K), jnp.int32),
            jax.ShapeDtypeStruct((NPAD, D), jnp.float32),
            jax.ShapeDtypeStruct((NPAD, D), jnp.float32),
            jax.ShapeDtypeStruct((NPAD, K), jnp.int32),
        ],
        scratch_shapes=[pltpu.VMEM((RQ, 16, NPAD // 16), jnp.float32)],
    )(featp, featT, wtT, wpT, bsum)


def _slow_call(featp, featT):
    return pl.pallas_call(
        _slow_body,
        grid=(NPAD // RQ,),
        in_specs=[
            pl.BlockSpec((RQ, D), lambda i: (i, 0)),
            pl.BlockSpec((D, NPAD), lambda i: (0, 0)),
        ],
        out_specs=pl.BlockSpec((RQ, K), lambda i: (i, 0)),
        out_shape=jax.ShapeDtypeStruct((NPAD, K), jnp.int32),
        scratch_shapes=[pltpu.VMEM((RQ, 16, NPAD // 16), jnp.float32)],
    )(featp, featT)


def _sc_body(t0_hbm, base_hbm, idx_hbm, out_hbm, idxv, rows, basev, outv, sem):
    nc = plsc.get_sparse_core_info().num_cores
    wid = lax.axis_index("s") * nc + lax.axis_index("c")

    def body(g, carry):
        nb = wid * NPW + g * BN
        pltpu.sync_copy(idx_hbm.at[pl.ds(nb * K, BN * K)], idxv)
        pltpu.async_copy(t0_hbm.at[idxv], rows, sem).wait()
        pltpu.sync_copy(base_hbm.at[pl.ds(nb, BN)], basev)
        for b in range(BN):
            for c in range(D // 16):
                sl = pl.ds(c * 16, 16)
                acc = rows[b * K, sl]
                for j in range(1, K):
                    acc = jnp.minimum(acc, rows[b * K + j, sl])
                outv[b, sl] = basev[b, sl] - acc
        pltpu.sync_copy(outv, out_hbm.at[pl.ds(nb, BN)])
        return carry

    lax.fori_loop(0, NIT, body, 0)


@functools.cache
def _sc_gather_min():
    return pl.kernel(
        _sc_body,
        out_type=jax.ShapeDtypeStruct((NPAD, D), jnp.float32),
        mesh=plsc.VectorSubcoreMesh(core_axis_name="c", subcore_axis_name="s"),
        scratch_types=[
            pltpu.VMEM((BN * K,), jnp.int32),
            pltpu.VMEM((BN * K, D), jnp.float32),
            pltpu.VMEM((BN, D), jnp.float32),
            pltpu.VMEM((BN, D), jnp.float32),
            pltpu.SemaphoreType.DMA,
        ],
    )


@jax.jit
def kernel(feat, W_theta, b_theta, W_phi, b_phi):
    featp = jnp.pad(feat, ((0, NPAD - N), (0, 0)))
    featT = featp.T
    bsum = (b_theta + b_phi).reshape(1, D)
    idx, t0, base, ovf = _knn_call(featp, featT, W_theta.T, W_phi.T, bsum)
    idx = lax.cond(jnp.any(ovf[:N, 0] != 0),
                   lambda: _slow_call(featp, featT),
                   lambda: idx)
    out = _sc_gather_min()(t0, base, idx.reshape(-1))
    return out[:N]


# speedup vs baseline: 4.9238x; 1.5436x over previous
import functools

import jax
import jax.numpy as jnp
from jax import lax
from jax.experimental import pallas as pl
from jax.experimental.pallas import tpu as pltpu
from jax.experimental.pallas import tpu_sc as plsc

N = 10000
D = 128
K = 16
NPAD = 10240
RQ = 256

NW = 32
NPW = NPAD // NW
BN = 8
NIT = NPW // BN


def _knn_body(featq_ref, featT_ref, wtT_ref, wpT_ref, bsum_ref,
              idx_ref, t0_ref, base_ref, dist_ref):
    q = featq_ref[...]
    kT = featT_ref[...]
    t0 = jnp.dot(q, wtT_ref[...], preferred_element_type=jnp.float32)
    ph = jnp.dot(q, wpT_ref[...], preferred_element_type=jnp.float32)
    t0_ref[...] = t0
    base_ref[...] = t0 + ph + bsum_ref[...]

    sqk = jnp.sum(kT * kT, axis=0, keepdims=True)
    colv = lax.broadcasted_iota(jnp.int32, (1, NPAD), 1)
    d = sqk - 2.0 * jnp.dot(q, kT, preferred_element_type=jnp.float32)
    dist_ref[...] = jnp.where(colv >= N, jnp.float32(1e30), d)

    cols = lax.broadcasted_iota(jnp.int32, (RQ, NPAD), 1)
    m = jnp.full((RQ, 1), -jnp.inf, jnp.float32)
    pi = jnp.full((RQ, 1), -1, jnp.int32)
    picks = []
    for _ in range(K):
        dd = dist_ref[...]
        ok = (dd > m) | ((dd == m) & (cols > pi))
        mt = jnp.min(jnp.where(ok, dd, jnp.float32(jnp.inf)),
                     axis=1, keepdims=True)
        it = jnp.min(jnp.where(ok & (dd == mt), cols, jnp.int32(2 ** 30)),
                     axis=1, keepdims=True)
        picks.append(it)
        m, pi = mt, it
    idx_ref[...] = jnp.concatenate(picks, axis=1)


def _knn_call(featp, featT, wtT, wpT, bsum):
    return pl.pallas_call(
        _knn_body,
        grid=(NPAD // RQ,),
        in_specs=[
            pl.BlockSpec((RQ, D), lambda i: (i, 0)),
            pl.BlockSpec((D, NPAD), lambda i: (0, 0)),
            pl.BlockSpec((D, D), lambda i: (0, 0)),
            pl.BlockSpec((D, D), lambda i: (0, 0)),
            pl.BlockSpec((1, D), lambda i: (0, 0)),
        ],
        out_specs=[
            pl.BlockSpec((RQ, K), lambda i: (i, 0)),
            pl.BlockSpec((RQ, D), lambda i: (i, 0)),
            pl.BlockSpec((RQ, D), lambda i: (i, 0)),
        ],
        out_shape=[
            jax.ShapeDtypeStruct((NPAD, K), jnp.int32),
            jax.ShapeDtypeStruct((NPAD, D), jnp.float32),
            jax.ShapeDtypeStruct((NPAD, D), jnp.float32),
        ],
        scratch_shapes=[pltpu.VMEM((RQ, NPAD), jnp.float32)],
    )(featp, featT, wtT, wpT, bsum)


def _sc_body(t0_hbm, base_hbm, idx_hbm, out_hbm, idxv, rows, basev, outv, sem):
    nc = plsc.get_sparse_core_info().num_cores
    wid = lax.axis_index("s") * nc + lax.axis_index("c")

    def body(g, carry):
        nb = wid * NPW + g * BN
        pltpu.sync_copy(idx_hbm.at[pl.ds(nb * K, BN * K)], idxv)
        pltpu.async_copy(t0_hbm.at[idxv], rows, sem).wait()
        pltpu.sync_copy(base_hbm.at[pl.ds(nb, BN)], basev)
        for b in range(BN):
            for c in range(D // 16):
                sl = pl.ds(c * 16, 16)
                acc = rows[b * K, sl]
                for j in range(1, K):
                    acc = jnp.minimum(acc, rows[b * K + j, sl])
                outv[b, sl] = basev[b, sl] - acc
        pltpu.sync_copy(outv, out_hbm.at[pl.ds(nb, BN)])
        return carry

    lax.fori_loop(0, NIT, body, 0)


@functools.cache
def _sc_gather_min():
    return pl.kernel(
        _sc_body,
        out_type=jax.ShapeDtypeStruct((NPAD, D), jnp.float32),
        mesh=plsc.VectorSubcoreMesh(core_axis_name="c", subcore_axis_name="s"),
        scratch_types=[
            pltpu.VMEM((BN * K,), jnp.int32),
            pltpu.VMEM((BN * K, D), jnp.float32),
            pltpu.VMEM((BN, D), jnp.float32),
            pltpu.VMEM((BN, D), jnp.float32),
            pltpu.SemaphoreType.DMA,
        ],
    )


@jax.jit
def kernel(feat, W_theta, b_theta, W_phi, b_phi):
    featp = jnp.pad(feat, ((0, NPAD - N), (0, 0)))
    featT = featp.T
    bsum = (b_theta + b_phi).reshape(1, D)
    idx, t0, base = _knn_call(featp, featT, W_theta.T, W_phi.T, bsum)
    out = _sc_gather_min()(t0, base, idx.reshape(-1))
    return out[:N]
